# baseline (speedup 1.0000x reference)
import functools

import jax
import jax.numpy as jnp
from jax.experimental import pallas as pl
from jax.experimental.pallas import tpu as pltpu


def _gate_body(x_ref, gw_ref, nw_ref, noise_ref,
               a1_ref, a2_ref, w1_ref, w2_ref, lb_ref):
    x = x_ref[...]
    gw = gw_ref[...]
    logits = jax.lax.dot_general(
        x, gw, (((1,), (1,)), ((), ())), preferred_element_type=jnp.float32)
    T, E = logits.shape
    noisy = logits + noise_ref[...] * nw_ref[...]

    m1 = jnp.max(noisy, axis=1, keepdims=True)
    a1 = jnp.argmax(noisy, axis=1)
    lane = jax.lax.broadcasted_iota(jnp.int32, (T, E), 1)
    masked = jnp.where(lane == a1[:, None], -jnp.inf, noisy)
    m2 = jnp.max(masked, axis=1, keepdims=True)
    a2 = jnp.argmax(masked, axis=1)

    e2 = jnp.exp(m2 - m1)
    w1 = 1.0 / (1.0 + e2)
    w2 = 1.0 - w1

    a1_ref[...] = a1[:, None]
    a2_ref[...] = a2[:, None]
    w1_ref[...] = w1
    w2_ref[...] = w2

    p = jax.nn.softmax(logits, axis=1)
    gw_mean = jnp.mean(p, axis=0, keepdims=True)
    lb = jnp.mean((gw_mean - 1.0 / E) ** 2) * 0.01
    lb_ref[...] = jnp.broadcast_to(lb, (1, 1))


def _gate(x, gate_W, noise_weight, noise):
    T, _ = x.shape
    E = gate_W.shape[0]
    out_shapes = (
        jax.ShapeDtypeStruct((T, 1), jnp.int32),
        jax.ShapeDtypeStruct((T, 1), jnp.int32),
        jax.ShapeDtypeStruct((T, 1), jnp.float32),
        jax.ShapeDtypeStruct((T, 1), jnp.float32),
        jax.ShapeDtypeStruct((1, 1), jnp.float32),
    )
    return pl.pallas_call(_gate_body, out_shape=out_shapes)(
        x, gate_W, noise_weight.reshape(1, E), noise)


def _moe_body(nh, be_ref, nbu_ref,
              xs_ref, w1_ref, b1_ref, w2_ref, b2_ref, wp_ref, bp_ref, wgt_ref,
              ys_ref, acc_ref):
    h = pl.program_id(0)
    b = pl.program_id(1)

    @pl.when(b < nbu_ref[0])
    def _():
        x = xs_ref[...]
        h1 = jax.lax.dot_general(
            x, w1_ref[0], (((1,), (1,)), ((), ())),
            preferred_element_type=jnp.float32,
            precision=jax.lax.Precision.DEFAULT) + b1_ref[0]
        h2 = jax.lax.dot_general(
            x, w2_ref[0], (((1,), (1,)), ((), ())),
            preferred_element_type=jnp.float32,
            precision=jax.lax.Precision.DEFAULT) + b2_ref[0]
        s = h1 * (h2 * jax.lax.logistic(h2))
        yp = jax.lax.dot_general(
            s, wp_ref[0], (((1,), (1,)), ((), ())),
            preferred_element_type=jnp.float32,
            precision=jax.lax.Precision.DEFAULT)

        @pl.when(h == 0)
        def _():
            acc_ref[b] = yp

        @pl.when(h > 0)
        def _():
            acc_ref[b] += yp

        @pl.when(h == nh - 1)
        def _():
            ys_ref[...] = (acc_ref[b] + bp_ref[0]) * wgt_ref[...]


def _grouped_experts(x_sorted, wgt_col, block_expert, nb_used,
                     W1, b1, W2, b2, Wp, bp, BT, HB):
    R, _ = x_sorted.shape
    E, H, D = W1.shape
    NB = R // BT
    NH = H // HB

    def ys_idx(h, b, be, nbu):
        return (jnp.where(h == NH - 1, b, 0), 0)

    grid_spec = pltpu.PrefetchScalarGridSpec(
        num_scalar_prefetch=2,
        grid=(NH, NB),
        in_specs=[
            pl.BlockSpec((BT, D), lambda h, b, be, nbu: (b, 0)),
            pl.BlockSpec((1, HB, D), lambda h, b, be, nbu: (be[b], h, 0)),
            pl.BlockSpec((1, 1, HB), lambda h, b, be, nbu: (be[b], 0, h)),
            pl.BlockSpec((1, HB, D), lambda h, b, be, nbu: (be[b], h, 0)),
            pl.BlockSpec((1, 1, HB), lambda h, b, be, nbu: (be[b], 0, h)),
            pl.BlockSpec((1, D, HB), lambda h, b, be, nbu: (be[b], 0, h)),
            pl.BlockSpec((1, 1, D), lambda h, b, be, nbu: (be[b], 0, 0)),
            pl.BlockSpec((BT, 1), lambda h, b, be, nbu: (b, 0)),
        ],
        out_specs=pl.BlockSpec((BT, D), ys_idx),
        scratch_shapes=[pltpu.VMEM((NB, BT, D), jnp.float32)],
    )
    return pl.pallas_call(
        functools.partial(_moe_body, NH),
        grid_spec=grid_spec,
        out_shape=jax.ShapeDtypeStruct((R, D), jnp.float32),
        compiler_params=pltpu.CompilerParams(
            dimension_semantics=("arbitrary", "arbitrary")),
    )(block_expert, nb_used, x_sorted,
      W1, b1[:, None, :], W2, b2[:, None, :], Wp, bp[:, None, :], wgt_col)


def kernel(x_flat, gate_W, noise_weight, noise, W1, b1, W2, b2, Wp, bp):
    T, D = x_flat.shape
    E, H, _ = W1.shape
    K = 2
    A = T * K
    BT = min(256, T)
    HB = min(1024, H)
    NB = A // BT + E
    R = NB * BT

    a1, a2, w1, w2, lb = _gate(x_flat, gate_W, noise_weight, noise)
    lb_loss = lb[0, 0]

    eflat = jnp.concatenate([a1[:, 0], a2[:, 0]])
    wflat = jnp.concatenate([w1[:, 0], w2[:, 0]])
    tokflat = jnp.tile(jnp.arange(T, dtype=jnp.int32), (K,))

    onehot = (eflat[:, None] == jnp.arange(E, dtype=jnp.int32)[None, :])
    ranks = jnp.cumsum(onehot.astype(jnp.int32), axis=0)
    counts = ranks[-1]
    pcounts = ((counts + BT - 1) // BT) * BT
    poffsets = jnp.concatenate(
        [jnp.zeros((1,), counts.dtype), jnp.cumsum(pcounts)[:-1]])

    rank_j = jnp.sum(ranks * onehot, axis=1)
    pp = (poffsets[eflat] + rank_j - 1).astype(jnp.int32)

    tok = jnp.zeros((R,), jnp.int32).at[pp].set(tokflat)
    wgt = jnp.zeros((R,), jnp.float32).at[pp].set(wflat)
    pos0, pos1 = pp[:T], pp[T:]

    total_padded = jnp.sum(pcounts)
    nb_used = (total_padded // BT).astype(jnp.int32)[None]
    block_starts = jnp.arange(NB) * BT
    block_expert = jnp.clip(
        jnp.searchsorted(poffsets, block_starts, side="right") - 1,
        0, E - 1).astype(jnp.int32)

    x_sorted = jnp.take(x_flat, tok, axis=0)
    ys = _grouped_experts(x_sorted, wgt[:, None], block_expert, nb_used,
                          W1, b1, W2, b2, Wp, bp, BT, HB)
    out = jnp.take(ys, pos0, axis=0) + jnp.take(ys, pos1, axis=0)
    return out, lb_loss


# speedup vs baseline: 1.0235x; 1.0235x over previous
import functools

import jax
import jax.numpy as jnp
from jax.experimental import pallas as pl
from jax.experimental.pallas import tpu as pltpu


def _gate_body(BT, NB, x_ref, gw_ref, nw_ref, noise_ref,
               pp_ref, wf_ref, be_ref, nbu_ref, lb_ref):
    x = x_ref[...]
    gw = gw_ref[...]
    logits = jax.lax.dot_general(
        x, gw, (((1,), (1,)), ((), ())), preferred_element_type=jnp.float32)
    T, E = logits.shape
    noisy = logits + noise_ref[...] * nw_ref[...]

    m1 = jnp.max(noisy, axis=1, keepdims=True)
    a1 = jnp.argmax(noisy, axis=1)
    lane = jax.lax.broadcasted_iota(jnp.int32, (T, E), 1)
    masked = jnp.where(lane == a1[:, None], -jnp.inf, noisy)
    m2 = jnp.max(masked, axis=1, keepdims=True)
    a2 = jnp.argmax(masked, axis=1)

    e2 = jnp.exp(m2 - m1)
    w1 = 1.0 / (1.0 + e2)
    w2 = 1.0 - w1
    wf_ref[...] = jnp.concatenate([w1, w2], axis=0)

    oh = jnp.concatenate(
        [(lane == a1[:, None]), (lane == a2[:, None])], axis=0
    ).astype(jnp.float32)
    A = 2 * T
    CH = 512
    row = jax.lax.broadcasted_iota(jnp.int32, (CH, CH), 0)
    col = jax.lax.broadcasted_iota(jnp.int32, (CH, CH), 1)
    L = (row >= col).astype(jnp.float32)
    carry = jnp.zeros((1, E), jnp.float32)
    parts = []
    for c in range(A // CH):
        ohc = oh[c * CH:(c + 1) * CH]
        rc = jax.lax.dot_general(
            L, ohc, (((1,), (0,)), ((), ())),
            preferred_element_type=jnp.float32) + carry
        parts.append(rc)
        carry = rc[CH - 1:CH]
    ranks = jnp.concatenate(parts, axis=0)
    counts = carry

    pcounts = jnp.floor((counts + (BT - 1)) / BT) * BT
    er = jax.lax.broadcasted_iota(jnp.int32, (E, E), 0)
    ec = jax.lax.broadcasted_iota(jnp.int32, (E, E), 1)
    Mx = (er < ec).astype(jnp.float32)
    poffsets = jax.lax.dot_general(
        pcounts, Mx, (((1,), (0,)), ((), ())),
        preferred_element_type=jnp.float32)

    pp = (jnp.sum(oh * poffsets, axis=1, keepdims=True)
          + jnp.sum(oh * ranks, axis=1, keepdims=True) - 1.0)
    pp_ref[...] = pp.astype(jnp.int32)

    nbu = jnp.sum(pcounts) / BT
    nbu_ref[...] = jnp.broadcast_to(nbu, (1, 1)).astype(jnp.int32)

    bs = jax.lax.broadcasted_iota(jnp.int32, (NB, 1), 0).astype(
        jnp.float32) * BT
    be = jnp.sum((bs >= poffsets).astype(jnp.int32), axis=1, keepdims=True) - 1
    be_ref[...] = jnp.clip(be, 0, E - 1)

    p = jax.nn.softmax(logits, axis=1)
    gw_mean = jnp.mean(p, axis=0, keepdims=True)
    lb = jnp.mean((gw_mean - 1.0 / E) ** 2) * 0.01
    lb_ref[...] = jnp.broadcast_to(lb, (1, 1))


def _gate(x, gate_W, noise_weight, noise, BT, NB):
    T, _ = x.shape
    E = gate_W.shape[0]
    out_shapes = (
        jax.ShapeDtypeStruct((2 * T, 1), jnp.int32),
        jax.ShapeDtypeStruct((2 * T, 1), jnp.float32),
        jax.ShapeDtypeStruct((NB, 1), jnp.int32),
        jax.ShapeDtypeStruct((1, 1), jnp.int32),
        jax.ShapeDtypeStruct((1, 1), jnp.float32),
    )
    return pl.pallas_call(
        functools.partial(_gate_body, BT, NB), out_shape=out_shapes)(
        x, gate_W, noise_weight.reshape(1, E), noise)


def _moe_body(nh, be_ref, nbu_ref,
              xs_ref, w1_ref, b1_ref, w2_ref, b2_ref, wp_ref, bp_ref, wgt_ref,
              ys_ref, acc_ref):
    h = pl.program_id(0)
    b = pl.program_id(1)

    @pl.when(b < nbu_ref[0])
    def _():
        x = xs_ref[...]
        h1 = jax.lax.dot_general(
            x, w1_ref[0], (((1,), (1,)), ((), ())),
            preferred_element_type=jnp.float32,
            precision=jax.lax.Precision.DEFAULT) + b1_ref[0]
        h2 = jax.lax.dot_general(
            x, w2_ref[0], (((1,), (1,)), ((), ())),
            preferred_element_type=jnp.float32,
            precision=jax.lax.Precision.DEFAULT) + b2_ref[0]
        s = h1 * (h2 * jax.lax.logistic(h2))
        yp = jax.lax.dot_general(
            s, wp_ref[0], (((1,), (1,)), ((), ())),
            preferred_element_type=jnp.float32,
            precision=jax.lax.Precision.DEFAULT)

        @pl.when(h == 0)
        def _():
            acc_ref[b] = yp

        @pl.when(h > 0)
        def _():
            acc_ref[b] += yp

        @pl.when(h == nh - 1)
        def _():
            ys_ref[...] = (acc_ref[b] + bp_ref[0]) * wgt_ref[...]


def _grouped_experts(x_sorted, wgt_col, block_expert, nb_used,
                     W1, b1, W2, b2, Wp, bp, BT, HB):
    R, _ = x_sorted.shape
    E, H, D = W1.shape
    NB = R // BT
    NH = H // HB

    def ys_idx(h, b, be, nbu):
        return (jnp.where(h == NH - 1, b, 0), 0)

    grid_spec = pltpu.PrefetchScalarGridSpec(
        num_scalar_prefetch=2,
        grid=(NH, NB),
        in_specs=[
            pl.BlockSpec((BT, D), lambda h, b, be, nbu: (b, 0)),
            pl.BlockSpec((1, HB, D), lambda h, b, be, nbu: (be[b], h, 0)),
            pl.BlockSpec((1, 1, HB), lambda h, b, be, nbu: (be[b], 0, h)),
            pl.BlockSpec((1, HB, D), lambda h, b, be, nbu: (be[b], h, 0)),
            pl.BlockSpec((1, 1, HB), lambda h, b, be, nbu: (be[b], 0, h)),
            pl.BlockSpec((1, D, HB), lambda h, b, be, nbu: (be[b], 0, h)),
            pl.BlockSpec((1, 1, D), lambda h, b, be, nbu: (be[b], 0, 0)),
            pl.BlockSpec((BT, 1), lambda h, b, be, nbu: (b, 0)),
        ],
        out_specs=pl.BlockSpec((BT, D), ys_idx),
        scratch_shapes=[pltpu.VMEM((NB, BT, D), jnp.float32)],
    )
    return pl.pallas_call(
        functools.partial(_moe_body, NH),
        grid_spec=grid_spec,
        out_shape=jax.ShapeDtypeStruct((R, D), jnp.float32),
        compiler_params=pltpu.CompilerParams(
            dimension_semantics=("arbitrary", "arbitrary")),
    )(block_expert, nb_used, x_sorted,
      W1, b1[:, None, :], W2, b2[:, None, :], Wp, bp[:, None, :], wgt_col)


def kernel(x_flat, gate_W, noise_weight, noise, W1, b1, W2, b2, Wp, bp):
    T, D = x_flat.shape
    E, H, _ = W1.shape
    K = 2
    A = T * K
    BT = min(256, T)
    HB = min(1024, H)
    NB = A // BT + E
    R = NB * BT

    pp2, wf2, be2, nbu2, lb = _gate(x_flat, gate_W, noise_weight, noise,
                                    BT, NB)
    lb_loss = lb[0, 0]
    pp = pp2[:, 0]
    tokflat = jnp.tile(jnp.arange(T, dtype=jnp.int32), (K,))
    tok = jnp.zeros((R,), jnp.int32).at[pp].set(tokflat)
    wgt = jnp.zeros((R,), jnp.float32).at[pp].set(wf2[:, 0])
    pos0, pos1 = pp[:T], pp[T:]
    block_expert = be2[:, 0]
    nb_used = nbu2[0]

    x_sorted = jnp.take(x_flat, tok, axis=0)
    ys = _grouped_experts(x_sorted, wgt[:, None], block_expert, nb_used,
                          W1, b1, W2, b2, Wp, bp, BT, HB)
    out = jnp.take(ys, pos0, axis=0) + jnp.take(ys, pos1, axis=0)
    return out, lb_loss


# speedup vs baseline: 1.2115x; 1.1837x over previous
import functools

import jax
import jax.numpy as jnp
from jax.experimental import pallas as pl
from jax.experimental.pallas import tpu as pltpu
from jax.experimental.pallas import tpu_sc as plsc


def _gate_body(BT, NB, x_ref, gw_ref, nw_ref, noise_ref,
               pp_ref, wf_ref, be_ref, nbu_ref, lb_ref):
    x = x_ref[...]
    gw = gw_ref[...]
    logits = jax.lax.dot_general(
        x, gw, (((1,), (1,)), ((), ())), preferred_element_type=jnp.float32)
    T, E = logits.shape
    noisy = logits + noise_ref[...] * nw_ref[...]

    m1 = jnp.max(noisy, axis=1, keepdims=True)
    a1 = jnp.argmax(noisy, axis=1)
    lane = jax.lax.broadcasted_iota(jnp.int32, (T, E), 1)
    masked = jnp.where(lane == a1[:, None], -jnp.inf, noisy)
    m2 = jnp.max(masked, axis=1, keepdims=True)
    a2 = jnp.argmax(masked, axis=1)

    e2 = jnp.exp(m2 - m1)
    w1 = 1.0 / (1.0 + e2)
    w2 = 1.0 - w1
    wf_ref[...] = jnp.concatenate([w1, w2], axis=0)

    oh = jnp.concatenate(
        [(lane == a1[:, None]), (lane == a2[:, None])], axis=0
    ).astype(jnp.float32)
    A = 2 * T
    CH = 512
    row = jax.lax.broadcasted_iota(jnp.int32, (CH, CH), 0)
    col = jax.lax.broadcasted_iota(jnp.int32, (CH, CH), 1)
    L = (row >= col).astype(jnp.float32)
    carry = jnp.zeros((1, E), jnp.float32)
    parts = []
    for c in range(A // CH):
        ohc = oh[c * CH:(c + 1) * CH]
        rc = jax.lax.dot_general(
            L, ohc, (((1,), (0,)), ((), ())),
            preferred_element_type=jnp.float32) + carry
        parts.append(rc)
        carry = rc[CH - 1:CH]
    ranks = jnp.concatenate(parts, axis=0)
    counts = carry

    pcounts = jnp.floor((counts + (BT - 1)) / BT) * BT
    er = jax.lax.broadcasted_iota(jnp.int32, (E, E), 0)
    ec = jax.lax.broadcasted_iota(jnp.int32, (E, E), 1)
    Mx = (er < ec).astype(jnp.float32)
    poffsets = jax.lax.dot_general(
        pcounts, Mx, (((1,), (0,)), ((), ())),
        preferred_element_type=jnp.float32)

    pp = (jnp.sum(oh * poffsets, axis=1, keepdims=True)
          + jnp.sum(oh * ranks, axis=1, keepdims=True) - 1.0)
    pp_ref[...] = pp.astype(jnp.int32)

    nbu = jnp.sum(pcounts) / BT
    nbu_ref[...] = jnp.broadcast_to(nbu, (1, 1)).astype(jnp.int32)

    bs = jax.lax.broadcasted_iota(jnp.int32, (NB, 1), 0).astype(
        jnp.float32) * BT
    be = jnp.sum((bs >= poffsets).astype(jnp.int32), axis=1, keepdims=True) - 1
    be_ref[...] = jnp.clip(be, 0, E - 1)

    p = jax.nn.softmax(logits, axis=1)
    gw_mean = jnp.mean(p, axis=0, keepdims=True)
    lb = jnp.mean((gw_mean - 1.0 / E) ** 2) * 0.01
    lb_ref[...] = jnp.broadcast_to(lb, (1, 1))


def _gate(x, gate_W, noise_weight, noise, BT, NB):
    T, _ = x.shape
    E = gate_W.shape[0]
    out_shapes = (
        jax.ShapeDtypeStruct((2 * T, 1), jnp.int32),
        jax.ShapeDtypeStruct((2 * T, 1), jnp.float32),
        jax.ShapeDtypeStruct((NB, 1), jnp.int32),
        jax.ShapeDtypeStruct((1, 1), jnp.int32),
        jax.ShapeDtypeStruct((1, 1), jnp.float32),
    )
    return pl.pallas_call(
        functools.partial(_gate_body, BT, NB), out_shape=out_shapes)(
        x, gate_W, noise_weight.reshape(1, E), noise)


def _sc_dispatch(x_flat, pp3, R):
    T, D = x_flat.shape
    NW, nch, CH = pp3.shape
    mesh = plsc.VectorSubcoreMesh(core_axis_name="c", subcore_axis_name="s")

    @functools.partial(
        pl.kernel, mesh=mesh,
        out_type=jax.ShapeDtypeStruct((R, D), jnp.float32),
        scratch_types=[
            pltpu.VMEM((nch, CH), jnp.int32),
            pltpu.VMEM((CH, D), jnp.float32),
            pltpu.SemaphoreType.DMA,
        ],
    )
    def disp(x_hbm, pp_hbm, xs_hbm, idx_v, rows_v, sem):
        wid = jax.lax.axis_index("s") * 2 + jax.lax.axis_index("c")
        pltpu.sync_copy(pp_hbm.at[wid], idx_v)
        for c in range(nch):
            j0 = wid * (nch * CH) + c * CH
            src = jax.lax.rem(j0, T)
            pltpu.sync_copy(x_hbm.at[pl.ds(src, CH)], rows_v)
            pltpu.async_copy(rows_v, xs_hbm.at[idx_v.at[c]], sem).wait()

    return disp(x_flat, pp3)


def _moe_body(nh, be_ref, nbu_ref,
              xs_ref, w1_ref, b1_ref, w2_ref, b2_ref, wp_ref, bp_ref, wgt_ref,
              ys_ref, acc_ref):
    h = pl.program_id(0)
    b = pl.program_id(1)

    @pl.when(b < nbu_ref[0])
    def _():
        x = xs_ref[...]
        h1 = jax.lax.dot_general(
            x, w1_ref[0], (((1,), (1,)), ((), ())),
            preferred_element_type=jnp.float32,
            precision=jax.lax.Precision.DEFAULT) + b1_ref[0]
        h2 = jax.lax.dot_general(
            x, w2_ref[0], (((1,), (1,)), ((), ())),
            preferred_element_type=jnp.float32,
            precision=jax.lax.Precision.DEFAULT) + b2_ref[0]
        s = h1 * (h2 * jax.lax.logistic(h2))
        yp = jax.lax.dot_general(
            s, wp_ref[0], (((1,), (1,)), ((), ())),
            preferred_element_type=jnp.float32,
            precision=jax.lax.Precision.DEFAULT)

        @pl.when(h == 0)
        def _():
            acc_ref[b] = yp

        @pl.when(h > 0)
        def _():
            acc_ref[b] += yp

        @pl.when(h == nh - 1)
        def _():
            ys_ref[...] = (acc_ref[b] + bp_ref[0]) * wgt_ref[...]


def _grouped_experts(x_sorted, wgt_col, block_expert, nb_used,
                     W1, b1, W2, b2, Wp, bp, BT, HB):
    R, _ = x_sorted.shape
    E, H, D = W1.shape
    NB = R // BT
    NH = H // HB

    def ys_idx(h, b, be, nbu):
        return (jnp.where(h == NH - 1, b, 0), 0)

    grid_spec = pltpu.PrefetchScalarGridSpec(
        num_scalar_prefetch=2,
        grid=(NH, NB),
        in_specs=[
            pl.BlockSpec((BT, D), lambda h, b, be, nbu: (b, 0)),
            pl.BlockSpec((1, HB, D), lambda h, b, be, nbu: (be[b], h, 0)),
            pl.BlockSpec((1, 1, HB), lambda h, b, be, nbu: (be[b], 0, h)),
            pl.BlockSpec((1, HB, D), lambda h, b, be, nbu: (be[b], h, 0)),
            pl.BlockSpec((1, 1, HB), lambda h, b, be, nbu: (be[b], 0, h)),
            pl.BlockSpec((1, D, HB), lambda h, b, be, nbu: (be[b], 0, h)),
            pl.BlockSpec((1, 1, D), lambda h, b, be, nbu: (be[b], 0, 0)),
            pl.BlockSpec((BT, 1), lambda h, b, be, nbu: (b, 0)),
        ],
        out_specs=pl.BlockSpec((BT, D), ys_idx),
        scratch_shapes=[pltpu.VMEM((NB, BT, D), jnp.float32)],
    )
    return pl.pallas_call(
        functools.partial(_moe_body, NH),
        grid_spec=grid_spec,
        out_shape=jax.ShapeDtypeStruct((R, D), jnp.float32),
        compiler_params=pltpu.CompilerParams(
            dimension_semantics=("arbitrary", "arbitrary")),
    )(block_expert, nb_used, x_sorted,
      W1, b1[:, None, :], W2, b2[:, None, :], Wp, bp[:, None, :], wgt_col)


def kernel(x_flat, gate_W, noise_weight, noise, W1, b1, W2, b2, Wp, bp):
    T, D = x_flat.shape
    E, H, _ = W1.shape
    K = 2
    A = T * K
    BT = min(256, T)
    HB = min(1024, H)
    NB = A // BT + E
    R = NB * BT

    pp2, wf2, be2, nbu2, lb = _gate(x_flat, gate_W, noise_weight, noise,
                                    BT, NB)
    lb_loss = lb[0, 0]
    pp = pp2[:, 0]
    wgt = jnp.zeros((R,), jnp.float32).at[pp].set(wf2[:, 0])
    pos0, pos1 = pp[:T], pp[T:]
    block_expert = be2[:, 0]
    nb_used = nbu2[0]

    NW = 32
    per = A // NW
    CH = 64 if per % 64 == 0 else per
    x_sorted = _sc_dispatch(x_flat, pp2.reshape(NW, per // CH, CH), R)
    ys = _grouped_experts(x_sorted, wgt[:, None], block_expert, nb_used,
                          W1, b1, W2, b2, Wp, bp, BT, HB)
    out = jnp.take(ys, pos0, axis=0) + jnp.take(ys, pos1, axis=0)
    return out, lb_loss


# speedup vs baseline: 1.2215x; 1.0082x over previous
import functools

import jax
import jax.numpy as jnp
from jax.experimental import pallas as pl
from jax.experimental.pallas import tpu as pltpu
from jax.experimental.pallas import tpu_sc as plsc


def _gate_body(BT, NB, x_ref, gw_ref, nw_ref, noise_ref,
               pp_ref, wf_ref, be_ref, nbu_ref, lb_ref):
    x = x_ref[...]
    gw = gw_ref[...]
    logits = jax.lax.dot_general(
        x, gw, (((1,), (1,)), ((), ())), preferred_element_type=jnp.float32)
    T, E = logits.shape
    noisy = logits + noise_ref[...] * nw_ref[...]

    m1 = jnp.max(noisy, axis=1, keepdims=True)
    a1 = jnp.argmax(noisy, axis=1)
    lane = jax.lax.broadcasted_iota(jnp.int32, (T, E), 1)
    masked = jnp.where(lane == a1[:, None], -jnp.inf, noisy)
    m2 = jnp.max(masked, axis=1, keepdims=True)
    a2 = jnp.argmax(masked, axis=1)

    e2 = jnp.exp(m2 - m1)
    w1 = 1.0 / (1.0 + e2)
    w2 = 1.0 - w1
    wf_ref[...] = jnp.concatenate([w1, w2], axis=0)

    oh = jnp.concatenate(
        [(lane == a1[:, None]), (lane == a2[:, None])], axis=0
    ).astype(jnp.float32)
    A = 2 * T
    CH = 512
    row = jax.lax.broadcasted_iota(jnp.int32, (CH, CH), 0)
    col = jax.lax.broadcasted_iota(jnp.int32, (CH, CH), 1)
    L = (row >= col).astype(jnp.float32)
    carry = jnp.zeros((1, E), jnp.float32)
    parts = []
    for c in range(A // CH):
        ohc = oh[c * CH:(c + 1) * CH]
        rc = jax.lax.dot_general(
            L, ohc, (((1,), (0,)), ((), ())),
            preferred_element_type=jnp.float32) + carry
        parts.append(rc)
        carry = rc[CH - 1:CH]
    ranks = jnp.concatenate(parts, axis=0)
    counts = carry

    pcounts = jnp.floor((counts + (BT - 1)) / BT) * BT
    er = jax.lax.broadcasted_iota(jnp.int32, (E, E), 0)
    ec = jax.lax.broadcasted_iota(jnp.int32, (E, E), 1)
    Mx = (er < ec).astype(jnp.float32)
    poffsets = jax.lax.dot_general(
        pcounts, Mx, (((1,), (0,)), ((), ())),
        preferred_element_type=jnp.float32)

    pp = (jnp.sum(oh * poffsets, axis=1, keepdims=True)
          + jnp.sum(oh * ranks, axis=1, keepdims=True) - 1.0)
    pp_ref[...] = pp.astype(jnp.int32)

    nbu = jnp.sum(pcounts) / BT
    nbu_ref[...] = jnp.broadcast_to(nbu, (1, 1)).astype(jnp.int32)

    bs = jax.lax.broadcasted_iota(jnp.int32, (NB, 1), 0).astype(
        jnp.float32) * BT
    be = jnp.sum((bs >= poffsets).astype(jnp.int32), axis=1, keepdims=True) - 1
    be_ref[...] = jnp.clip(be, 0, E - 1)

    p = jax.nn.softmax(logits, axis=1)
    gw_mean = jnp.mean(p, axis=0, keepdims=True)
    lb = jnp.mean((gw_mean - 1.0 / E) ** 2) * 0.01
    lb_ref[...] = jnp.broadcast_to(lb, (1, 1))


def _gate(x, gate_W, noise_weight, noise, BT, NB):
    T, _ = x.shape
    E = gate_W.shape[0]
    out_shapes = (
        jax.ShapeDtypeStruct((2 * T, 1), jnp.int32),
        jax.ShapeDtypeStruct((2 * T, 1), jnp.float32),
        jax.ShapeDtypeStruct((NB, 1), jnp.int32),
        jax.ShapeDtypeStruct((1, 1), jnp.int32),
        jax.ShapeDtypeStruct((1, 1), jnp.float32),
    )
    return pl.pallas_call(
        functools.partial(_gate_body, BT, NB), out_shape=out_shapes)(
        x, gate_W, noise_weight.reshape(1, E), noise)


def _sc_dispatch(x_flat, pp3, R):
    T, D = x_flat.shape
    NW, nch, CH = pp3.shape
    mesh = plsc.VectorSubcoreMesh(core_axis_name="c", subcore_axis_name="s")

    @functools.partial(
        pl.kernel, mesh=mesh,
        out_type=jax.ShapeDtypeStruct((R, D), jnp.float32),
        scratch_types=[
            pltpu.VMEM((nch, CH), jnp.int32),
            pltpu.VMEM((CH, D), jnp.float32),
            pltpu.SemaphoreType.DMA,
        ],
    )
    def disp(x_hbm, pp_hbm, xs_hbm, idx_v, rows_v, sem):
        wid = jax.lax.axis_index("s") * 2 + jax.lax.axis_index("c")
        pltpu.sync_copy(pp_hbm.at[wid], idx_v)
        for c in range(nch):
            j0 = wid * (nch * CH) + c * CH
            src = jax.lax.rem(j0, T)
            pltpu.sync_copy(x_hbm.at[pl.ds(src, CH)], rows_v)
            pltpu.async_copy(rows_v, xs_hbm.at[idx_v.at[c]], sem).wait()

    return disp(x_flat, pp3)


def _moe_body(nh, be_ref, nbu_ref,
              xs_ref, w1_ref, b1_ref, w2_ref, b2_ref, wp_ref, bp_ref, wgt_ref,
              ys_ref, acc_ref):
    h = pl.program_id(0)
    b = pl.program_id(1)

    @pl.when(b < nbu_ref[0])
    def _():
        x = xs_ref[...]
        h1 = jax.lax.dot_general(
            x, w1_ref[0], (((1,), (1,)), ((), ())),
            preferred_element_type=jnp.float32,
            precision=jax.lax.Precision.DEFAULT) + b1_ref[0]
        h2 = jax.lax.dot_general(
            x, w2_ref[0], (((1,), (1,)), ((), ())),
            preferred_element_type=jnp.float32,
            precision=jax.lax.Precision.DEFAULT) + b2_ref[0]
        s = h1 * (h2 * jax.lax.logistic(h2))
        yp = jax.lax.dot_general(
            s, wp_ref[0], (((1,), (1,)), ((), ())),
            preferred_element_type=jnp.float32,
            precision=jax.lax.Precision.DEFAULT)

        @pl.when(h == 0)
        def _():
            acc_ref[b] = yp

        @pl.when(h > 0)
        def _():
            acc_ref[b] += yp

        @pl.when(h == nh - 1)
        def _():
            ys_ref[...] = (acc_ref[b] + bp_ref[0]) * wgt_ref[...]


def _grouped_experts(x_sorted, wgt_col, block_expert, nb_used,
                     W1, b1, W2, b2, Wp, bp, BT, HB):
    R, _ = x_sorted.shape
    E, H, D = W1.shape
    NB = R // BT
    NH = H // HB

    def ys_idx(h, b, be, nbu):
        return (jnp.where(h == NH - 1, b, 0), 0)

    grid_spec = pltpu.PrefetchScalarGridSpec(
        num_scalar_prefetch=2,
        grid=(NH, NB),
        in_specs=[
            pl.BlockSpec((BT, D), lambda h, b, be, nbu: (b, 0)),
            pl.BlockSpec((1, HB, D), lambda h, b, be, nbu: (be[b], h, 0)),
            pl.BlockSpec((1, 1, HB), lambda h, b, be, nbu: (be[b], 0, h)),
            pl.BlockSpec((1, HB, D), lambda h, b, be, nbu: (be[b], h, 0)),
            pl.BlockSpec((1, 1, HB), lambda h, b, be, nbu: (be[b], 0, h)),
            pl.BlockSpec((1, D, HB), lambda h, b, be, nbu: (be[b], 0, h)),
            pl.BlockSpec((1, 1, D), lambda h, b, be, nbu: (be[b], 0, 0)),
            pl.BlockSpec((BT, 1), lambda h, b, be, nbu: (b, 0)),
        ],
        out_specs=pl.BlockSpec((BT, D), ys_idx),
        scratch_shapes=[pltpu.VMEM((NB, BT, D), jnp.float32)],
    )
    return pl.pallas_call(
        functools.partial(_moe_body, NH),
        grid_spec=grid_spec,
        out_shape=jax.ShapeDtypeStruct((R, D), jnp.float32),
        compiler_params=pltpu.CompilerParams(
            dimension_semantics=("arbitrary", "arbitrary")),
    )(block_expert, nb_used, x_sorted,
      W1, b1[:, None, :], W2, b2[:, None, :], Wp, bp[:, None, :], wgt_col)


def kernel(x_flat, gate_W, noise_weight, noise, W1, b1, W2, b2, Wp, bp):
    T, D = x_flat.shape
    E, H, _ = W1.shape
    K = 2
    A = T * K
    BT = min(256, T)
    HB = min(1024, H)
    NB = A // BT + E
    R = NB * BT

    pp2, wf2, be2, nbu2, lb = _gate(x_flat, gate_W, noise_weight, noise,
                                    BT, NB)
    lb_loss = lb[0, 0]
    pp = pp2[:, 0]
    wgt = jnp.zeros((R,), jnp.float32).at[pp].set(wf2[:, 0])
    block_expert = be2[:, 0]
    nb_used = nbu2[0]

    NW = 32
    per = A // NW
    CH = 64 if per % 64 == 0 else per
    x_sorted = _sc_dispatch(x_flat, pp2.reshape(NW, per // CH, CH), R)
    ys = _grouped_experts(x_sorted, wgt[:, None], block_expert, nb_used,
                          W1, b1, W2, b2, Wp, bp, BT, HB)
    out = jnp.take(ys, pp, axis=0).reshape(K, T, D).sum(axis=0)
    return out, lb_loss
